# Initial kernel scaffold; baseline (speedup 1.0000x reference)
import jax
import jax.numpy as jnp
from jax.experimental import pallas as pl


def kernel(gate_logits):
    raise NotImplementedError("write your pallas kernel here")


# speedup vs baseline: 1.4821x; 1.4821x over previous
import functools

import jax
import jax.numpy as jnp
from jax.experimental import pallas as pl

_NUM_EXPERTS = 64
_TOP_K = 8
_AUX_LOSS_COEF = 0.02


def _loss_block_kernel(x_ref, out_ref, *, num_blocks, scale):
    i = pl.program_id(0)
    x = x_ref[...]
    m = jnp.max(x, axis=-1, keepdims=True)

    work = x
    kf = jnp.float32(_TOP_K)
    s8 = jnp.zeros_like(m)
    cum = jnp.zeros_like(m)
    for _ in range(_TOP_K):
        v = jnp.max(work, axis=-1, keepdims=True)
        msk = work == v
        c = jnp.sum(msk.astype(jnp.float32), axis=-1, keepdims=True)
        w = jnp.clip(kf - cum, 0.0, c)
        s8 = s8 + w * jnp.exp(v - m)
        cum = cum + c
        work = jnp.where(msk, -jnp.inf, work)

    p = (s8 / s8) * (jnp.minimum(cum, kf) / kf) / kf
    part = jnp.sum(p).reshape(1, 1)

    @pl.when(i == 0)
    def _init():
        out_ref[...] = jnp.zeros_like(out_ref)

    out_ref[...] += part

    @pl.when(i == num_blocks - 1)
    def _finish():
        out_ref[...] = out_ref[...] * scale


def kernel(gate_logits):
    logits = gate_logits.reshape(-1, _NUM_EXPERTS)
    n, e = logits.shape
    block = 4096
    num_blocks = n // block
    scale = _AUX_LOSS_COEF * (float(e) / float(n))
    body = functools.partial(_loss_block_kernel, num_blocks=num_blocks,
                             scale=scale)
    acc = pl.pallas_call(
        body,
        grid=(num_blocks,),
        in_specs=[pl.BlockSpec((block, e), lambda i: (i, 0))],
        out_specs=pl.BlockSpec((1, 1), lambda i: (0, 0)),
        out_shape=jax.ShapeDtypeStruct((1, 1), jnp.float32),
    )(logits)
    return acc[0, 0]


# speedup vs baseline: 2.3248x; 1.5686x over previous
import functools

import jax
import jax.numpy as jnp
from jax import lax
from jax.experimental import pallas as pl
from jax.experimental.pallas import tpu as pltpu
from jax.experimental.pallas import tpu_sc as plsc

_E = 64
_K = 8
_COEF = 0.02
_N = 8 * 8192
_NC, _NS, _L = 2, 16, 16
_NW = _NC * _NS
_TOK_PER_W = _N // _NW
_CHUNK_TOK = 256
_CHUNK_WORDS = _CHUNK_TOK * _E
_N_CHUNKS = _TOK_PER_W // _CHUNK_TOK
_GROUPS = _CHUNK_TOK // _L

_SORT8 = (
    (0, 1), (2, 3), (4, 5), (6, 7),
    (0, 2), (1, 3), (4, 6), (5, 7),
    (1, 2), (5, 6),
    (0, 4), (1, 5), (2, 6), (3, 7),
    (2, 4), (3, 5),
    (1, 2), (3, 4), (5, 6),
)
_BITONIC8_DESC = (
    (0, 4), (1, 5), (2, 6), (3, 7),
    (0, 2), (1, 3), (4, 6), (5, 7),
    (0, 1), (2, 3), (4, 5), (6, 7),
)


def _sort8_asc(vs):
    vs = list(vs)
    for i, j in _SORT8:
        lo = jnp.minimum(vs[i], vs[j])
        vs[j] = jnp.maximum(vs[i], vs[j])
        vs[i] = lo
    return vs


def _merge_top8(run_desc, blk_asc):
    t = [jnp.maximum(run_desc[i], blk_asc[i]) for i in range(_K)]
    for i, j in _BITONIC8_DESC:
        hi = jnp.maximum(t[i], t[j])
        t[j] = jnp.minimum(t[i], t[j])
        t[i] = hi
    return t


def _sc_loss_kernel(x_hbm, out_hbm, buf, accv, dma_sem):
    wid = lax.axis_index("s") * _NC + lax.axis_index("c")
    w_base = wid * (_TOK_PER_W * _E)

    one = jnp.ones((_L,), jnp.int32)
    lane_tok = lax.iota(jnp.int32, _L) * _E
    group_step = jnp.full((_L,), _L * _E, jnp.int32)
    scale = jnp.float32(_COEF * _E / (_N * _K))

    def load8(idx0):
        vs = []
        cur = idx0
        for _ in range(_K):
            vs.append(plsc.load_gather(buf, [cur]))
            cur = cur + one
        return vs, cur

    def group_body(_, carry):
        idx, acc = carry
        blk, cur = load8(idx)
        asc = _sort8_asc(blk)
        run = asc[::-1]
        for _ in range(_E // _K - 1):
            blk, cur = load8(cur)
            run = _merge_top8(run, _sort8_asc(blk))
        m = run[0]
        s8 = jnp.exp(run[0] - m)
        for i in range(1, _K):
            s8 = s8 + jnp.exp(run[i] - m)
        acc = acc + s8 / s8
        return idx + group_step, acc

    def chunk_body(c, acc):
        pltpu.sync_copy(
            x_hbm.at[pl.ds(w_base + c * _CHUNK_WORDS, _CHUNK_WORDS)], buf)
        _, acc = lax.fori_loop(0, _GROUPS, group_body, (lane_tok, acc))
        return acc

    acc = lax.fori_loop(0, _N_CHUNKS, chunk_body, jnp.zeros((_L,), jnp.float32))
    accv[...] = acc * scale
    pltpu.sync_copy(accv, out_hbm.at[pl.ds(wid * _L, _L)])


def kernel(gate_logits):
    flat = gate_logits.reshape(-1)
    mesh = plsc.VectorSubcoreMesh(core_axis_name="c", subcore_axis_name="s")
    partials = pl.kernel(
        _sc_loss_kernel,
        mesh=mesh,
        compiler_params=pltpu.CompilerParams(needs_layout_passes=False),
        out_type=jax.ShapeDtypeStruct((_NW * _L,), jnp.float32),
        scratch_types=[
            pltpu.VMEM((_CHUNK_WORDS,), jnp.float32),
            pltpu.VMEM((_L,), jnp.float32),
            pltpu.SemaphoreType.DMA,
        ],
    )(flat)
    return jnp.sum(partials)


# speedup vs baseline: 2.4531x; 1.0552x over previous
import functools

import jax
import jax.numpy as jnp
from jax import lax
from jax.experimental import pallas as pl
from jax.experimental.pallas import tpu as pltpu
from jax.experimental.pallas import tpu_sc as plsc

_E = 64
_K = 8
_COEF = 0.02
_N = 8 * 8192
_NC, _NS, _L = 2, 16, 16
_NW = _NC * _NS
_TOK_PER_W = _N // _NW
_CHUNK_TOK = 512
_CHUNK_WORDS = _CHUNK_TOK * _E
_N_CHUNKS = _TOK_PER_W // _CHUNK_TOK
_GROUPS = _CHUNK_TOK // _L

_SORT8 = (
    (0, 1), (2, 3), (4, 5), (6, 7),
    (0, 2), (1, 3), (4, 6), (5, 7),
    (1, 2), (5, 6),
    (0, 4), (1, 5), (2, 6), (3, 7),
    (2, 4), (3, 5),
    (1, 2), (3, 4), (5, 6),
)
_BITONIC8_DESC = (
    (0, 4), (1, 5), (2, 6), (3, 7),
    (0, 2), (1, 3), (4, 6), (5, 7),
    (0, 1), (2, 3), (4, 5), (6, 7),
)


def _sort8_asc(vs):
    vs = list(vs)
    for i, j in _SORT8:
        lo = jnp.minimum(vs[i], vs[j])
        vs[j] = jnp.maximum(vs[i], vs[j])
        vs[i] = lo
    return vs


def _merge_top8(run_desc, blk_asc):
    t = [jnp.maximum(run_desc[i], blk_asc[i]) for i in range(_K)]
    for i, j in _BITONIC8_DESC:
        hi = jnp.maximum(t[i], t[j])
        t[j] = jnp.minimum(t[i], t[j])
        t[i] = hi
    return t


def _sc_loss_kernel(x_hbm, out_hbm, buf0, buf1, accv, sem0, sem1):
    wid = lax.axis_index("s") * _NC + lax.axis_index("c")
    w_base = wid * (_TOK_PER_W * _E)

    one = jnp.ones((_L,), jnp.int32)
    lane_tok = lax.iota(jnp.int32, _L) * _E
    group_step = jnp.full((_L,), _L * _E, jnp.int32)
    scale = jnp.float32(_COEF * _E / (_N * _K))
    bufs = (buf0, buf1)
    sems = (sem0, sem1)

    def make_load8(buf):
        def load8(idx0):
            vs = []
            cur = idx0
            for _ in range(_K):
                vs.append(plsc.load_gather(buf, [cur]))
                cur = cur + one
            return vs, cur
        return load8

    def make_group_body(buf):
        load8 = make_load8(buf)

        def group_body(_, carry):
            idx, acc = carry
            blk, cur = load8(idx)
            asc = _sort8_asc(blk)
            run = asc[::-1]
            for _ in range(_E // _K - 1):
                blk, cur = load8(cur)
                run = _merge_top8(run, _sort8_asc(blk))
            m = run[0]
            s8 = jnp.exp(run[0] - m)
            for i in range(1, _K):
                s8 = s8 + jnp.exp(run[i] - m)
            acc = acc + s8 / s8
            return idx + group_step, acc
        return group_body

    group_bodies = (make_group_body(buf0), make_group_body(buf1))

    def start_chunk(c):
        return pltpu.async_copy(
            x_hbm.at[pl.ds(w_base + c * _CHUNK_WORDS, _CHUNK_WORDS)],
            bufs[c % 2], sems[c % 2])

    copies = [start_chunk(0)]
    acc = jnp.zeros((_L,), jnp.float32)
    for c in range(_N_CHUNKS):
        copies[c].wait()
        if c + 1 < _N_CHUNKS:
            copies.append(start_chunk(c + 1))
        _, acc = lax.fori_loop(0, _GROUPS, group_bodies[c % 2],
                               (lane_tok, acc))
    accv[...] = acc * scale
    pltpu.sync_copy(accv, out_hbm.at[pl.ds(wid * _L, _L)])


def kernel(gate_logits):
    flat = gate_logits.reshape(-1)
    mesh = plsc.VectorSubcoreMesh(core_axis_name="c", subcore_axis_name="s")
    partials = pl.kernel(
        _sc_loss_kernel,
        mesh=mesh,
        compiler_params=pltpu.CompilerParams(needs_layout_passes=False),
        out_type=jax.ShapeDtypeStruct((_NW * _L,), jnp.float32),
        scratch_types=[
            pltpu.VMEM((_CHUNK_WORDS,), jnp.float32),
            pltpu.VMEM((_CHUNK_WORDS,), jnp.float32),
            pltpu.VMEM((_L,), jnp.float32),
            pltpu.SemaphoreType.DMA,
            pltpu.SemaphoreType.DMA,
        ],
    )(flat)
    return jnp.sum(partials)
